# baseline (speedup 1.0000x reference)
import jax
import jax.numpy as jnp
from jax.experimental import pallas as pl
from jax.experimental.pallas import tpu as pltpu

N = 4096
F = 256
H = 128
TAU = 0.5


def _mega_kernel(x1_ref, x2_ref, adj1_ref, adj2_ref, clm_ref,
                 w11_ref, b11_ref, w12_ref, b12_ref,
                 w21_ref, b21_ref, w22_ref, b22_ref,
                 z1_ref, z2_ref, loss_ref,
                 adj_scr, sup1_scr, sup2_scr, s2a_scr, s2b_scr,
                 z1b_scr, r1_scr, rs_scr, ws_scr):
    p = pl.program_id(0)
    i = pl.program_id(1)
    bm = z1_ref.shape[0]
    ni = pl.num_programs(1)

    @pl.when(p == 0)
    def _():
        xb = x1_ref[...].astype(jnp.bfloat16)
        sup = jnp.dot(xb, w11_ref[...], preferred_element_type=jnp.float32)
        sup1_scr[pl.ds(i * bm, bm), :] = sup.astype(jnp.bfloat16)

    @pl.when(p == 1)
    def _():
        ab = adj1_ref[...].astype(jnp.bfloat16)
        adj_scr[pl.ds(i * bm, bm), :] = ab
        acc = jnp.dot(ab, sup1_scr[...], preferred_element_type=jnp.float32)
        h = jnp.maximum(acc + b11_ref[...], 0.0)
        s2 = jnp.dot(h.astype(jnp.bfloat16), w12_ref[...],
                     preferred_element_type=jnp.float32)
        s2a_scr[pl.ds(i * bm, bm), :] = s2.astype(jnp.bfloat16)
        xb = x2_ref[...].astype(jnp.bfloat16)
        sup = jnp.dot(xb, w21_ref[...], preferred_element_type=jnp.float32)
        sup2_scr[pl.ds(i * bm, bm), :] = sup.astype(jnp.bfloat16)

    @pl.when(p == 2)
    def _():
        a1 = adj_scr[pl.ds(i * bm, bm), :]
        z1 = jnp.dot(a1, s2a_scr[...],
                     preferred_element_type=jnp.float32) + b12_ref[...]
        z1_ref[...] = z1
        z1b_scr[pl.ds(i * bm, bm), :] = z1.astype(jnp.bfloat16)
        r1_scr[pl.ds(i * bm, bm), :] = jax.lax.rsqrt(
            jnp.sum(z1 * z1, axis=1, keepdims=True)) * (1.0 / TAU)
        ab = adj2_ref[...].astype(jnp.bfloat16)
        adj_scr[pl.ds(i * bm, bm), :] = ab
        acc = jnp.dot(ab, sup2_scr[...], preferred_element_type=jnp.float32)
        h = jnp.maximum(acc + b21_ref[...], 0.0)
        s2 = jnp.dot(h.astype(jnp.bfloat16), w22_ref[...],
                     preferred_element_type=jnp.float32)
        s2b_scr[pl.ds(i * bm, bm), :] = s2.astype(jnp.bfloat16)

    @pl.when(p == 3)
    def _():
        a2 = adj_scr[pl.ds(i * bm, bm), :]
        z2 = jnp.dot(a2, s2b_scr[...],
                     preferred_element_type=jnp.float32) + b22_ref[...]
        z2_ref[...] = z2
        z2b = z2.astype(jnp.bfloat16)
        r2 = jax.lax.rsqrt(jnp.sum(z2 * z2, axis=1, keepdims=True))
        s = jax.lax.dot_general(z1b_scr[...], z2b, (((1,), (1,)), ((), ())),
                                preferred_element_type=jnp.float32)
        pexp = jnp.exp(s * r1_scr[...] * r2.reshape(1, bm))
        prs = jnp.sum(pexp, axis=1, keepdims=True)
        pws = jnp.sum(pexp * clm_ref[...], axis=1, keepdims=True)

        @pl.when(i == 0)
        def _():
            rs_scr[...] = prs
            ws_scr[...] = pws

        @pl.when(i > 0)
        def _():
            rs_scr[...] += prs
            ws_scr[...] += pws

        @pl.when(i == ni - 1)
        def _():
            lsum = jnp.sum(jnp.log(rs_scr[...] + 1e-8) - jnp.log(ws_scr[...]))
            loss_ref[...] = jnp.full((1, 1), lsum * (1.0 / N),
                                     dtype=jnp.float32)


def _mega(x1, adj1, x2, adj2, clm,
          W11, b11, W12, b12, W21, b21, W22, b22, bm=128):
    ni = N // bm
    z1, z2, loss = pl.pallas_call(
        _mega_kernel,
        grid=(4, ni),
        in_specs=[
            pl.BlockSpec((bm, F),
                         lambda p, i: (jnp.where(p == 0, i, ni - 1), 0)),
            pl.BlockSpec((bm, F),
                         lambda p, i: (jnp.where(p < 1, 0,
                                                 jnp.where(p == 1, i,
                                                           ni - 1)), 0)),
            pl.BlockSpec((bm, N),
                         lambda p, i: (jnp.where(p < 1, 0,
                                                 jnp.where(p == 1, i,
                                                           ni - 1)), 0)),
            pl.BlockSpec((bm, N),
                         lambda p, i: (jnp.where(p < 2, 0,
                                                 jnp.where(p == 2, i,
                                                           ni - 1)), 0)),
            pl.BlockSpec((N, bm),
                         lambda p, i: (0, jnp.where(p < 3, 0, i))),
            pl.BlockSpec((F, F), lambda p, i: (0, 0)),
            pl.BlockSpec((1, F), lambda p, i: (0, 0)),
            pl.BlockSpec((F, H), lambda p, i: (0, 0)),
            pl.BlockSpec((1, H), lambda p, i: (0, 0)),
            pl.BlockSpec((F, F), lambda p, i: (0, 0)),
            pl.BlockSpec((1, F), lambda p, i: (0, 0)),
            pl.BlockSpec((F, H), lambda p, i: (0, 0)),
            pl.BlockSpec((1, H), lambda p, i: (0, 0)),
        ],
        out_specs=[
            pl.BlockSpec((bm, H),
                         lambda p, i: (jnp.where(p < 2, 0,
                                                 jnp.where(p == 2, i,
                                                           ni - 1)), 0)),
            pl.BlockSpec((bm, H),
                         lambda p, i: (jnp.where(p < 3, 0, i), 0)),
            pl.BlockSpec((1, 1), lambda p, i: (0, 0)),
        ],
        out_shape=[
            jax.ShapeDtypeStruct((N, H), jnp.float32),
            jax.ShapeDtypeStruct((N, H), jnp.float32),
            jax.ShapeDtypeStruct((1, 1), jnp.float32),
        ],
        scratch_shapes=[
            pltpu.VMEM((N, N), jnp.bfloat16),
            pltpu.VMEM((N, F), jnp.bfloat16),
            pltpu.VMEM((N, F), jnp.bfloat16),
            pltpu.VMEM((N, H), jnp.bfloat16),
            pltpu.VMEM((N, H), jnp.bfloat16),
            pltpu.VMEM((N, H), jnp.bfloat16),
            pltpu.VMEM((N, 1), jnp.float32),
            pltpu.VMEM((N, 1), jnp.float32),
            pltpu.VMEM((N, 1), jnp.float32),
        ],
        compiler_params=pltpu.CompilerParams(
            vmem_limit_bytes=120 * 1024 * 1024,
        ),
    )(x1, x2, adj1, adj2, clm,
      W11.astype(jnp.bfloat16), b11.reshape(1, F),
      W12.astype(jnp.bfloat16), b12.reshape(1, H),
      W21.astype(jnp.bfloat16), b21.reshape(1, F),
      W22.astype(jnp.bfloat16), b22.reshape(1, H))
    return z1, z2, loss.reshape(())


def kernel(x1, adj1, x2, adj2, clm, W11, b11, W12, b12, W21, b21, W22, b22):
    return _mega(x1, adj1, x2, adj2, clm,
                 W11, b11, W12, b12, W21, b21, W22, b22)


# speedup vs baseline: 1.4694x; 1.4694x over previous
import jax
import jax.numpy as jnp
from jax.experimental import pallas as pl
from jax.experimental.pallas import tpu as pltpu

N = 4096
F = 256
H = 128
TAU = 0.5


def _towers_kernel(x1_ref, x2_ref, adj1_ref, adj2_ref,
                   w11_ref, b11_ref, w12_ref, b12_ref,
                   w21_ref, b21_ref, w22_ref, b22_ref,
                   z1_ref, z2_ref,
                   adj_scr, sup1_scr, sup2_scr, s2a_scr, s2b_scr):
    p = pl.program_id(0)
    i = pl.program_id(1)
    bm = z1_ref.shape[0]

    @pl.when(p == 0)
    def _():
        xb1 = x1_ref[...].astype(jnp.bfloat16)
        sup = jnp.dot(xb1, w11_ref[...], preferred_element_type=jnp.float32)
        sup1_scr[pl.ds(i * bm, bm), :] = sup.astype(jnp.bfloat16)
        xb2 = x2_ref[...].astype(jnp.bfloat16)
        sup = jnp.dot(xb2, w21_ref[...], preferred_element_type=jnp.float32)
        sup2_scr[pl.ds(i * bm, bm), :] = sup.astype(jnp.bfloat16)

    @pl.when(p == 1)
    def _():
        ab = adj1_ref[...].astype(jnp.bfloat16)
        adj_scr[pl.ds(i * bm, bm), :] = ab
        acc = jnp.dot(ab, sup1_scr[...], preferred_element_type=jnp.float32)
        h = jnp.maximum(acc + b11_ref[...], 0.0)
        s2 = jnp.dot(h.astype(jnp.bfloat16), w12_ref[...],
                     preferred_element_type=jnp.float32)
        s2a_scr[pl.ds(i * bm, bm), :] = s2.astype(jnp.bfloat16)

    @pl.when(p == 2)
    def _():
        a1 = adj_scr[pl.ds(i * bm, bm), :]
        z1_ref[...] = jnp.dot(a1, s2a_scr[...],
                              preferred_element_type=jnp.float32) + b12_ref[...]
        ab = adj2_ref[...].astype(jnp.bfloat16)
        adj_scr[pl.ds(i * bm, bm), :] = ab
        acc = jnp.dot(ab, sup2_scr[...], preferred_element_type=jnp.float32)
        h = jnp.maximum(acc + b21_ref[...], 0.0)
        s2 = jnp.dot(h.astype(jnp.bfloat16), w22_ref[...],
                     preferred_element_type=jnp.float32)
        s2b_scr[pl.ds(i * bm, bm), :] = s2.astype(jnp.bfloat16)

    @pl.when(p == 3)
    def _():
        a2 = adj_scr[pl.ds(i * bm, bm), :]
        z2_ref[...] = jnp.dot(a2, s2b_scr[...],
                              preferred_element_type=jnp.float32) + b22_ref[...]


def _towers(x1, x2, adj1, adj2,
            W11, b11, W12, b12, W21, b21, W22, b22, bm=256):
    ni = N // bm
    z1, z2 = pl.pallas_call(
        _towers_kernel,
        grid=(4, ni),
        in_specs=[
            pl.BlockSpec((bm, F),
                         lambda p, i: (jnp.where(p == 0, i, ni - 1), 0)),
            pl.BlockSpec((bm, F),
                         lambda p, i: (jnp.where(p == 0, i, ni - 1), 0)),
            pl.BlockSpec((bm, N),
                         lambda p, i: (jnp.where(p < 1, 0,
                                                 jnp.where(p == 1, i,
                                                           ni - 1)), 0)),
            pl.BlockSpec((bm, N),
                         lambda p, i: (jnp.where(p < 2, 0,
                                                 jnp.where(p == 2, i,
                                                           ni - 1)), 0)),
            pl.BlockSpec((F, F), lambda p, i: (0, 0)),
            pl.BlockSpec((1, F), lambda p, i: (0, 0)),
            pl.BlockSpec((F, H), lambda p, i: (0, 0)),
            pl.BlockSpec((1, H), lambda p, i: (0, 0)),
            pl.BlockSpec((F, F), lambda p, i: (0, 0)),
            pl.BlockSpec((1, F), lambda p, i: (0, 0)),
            pl.BlockSpec((F, H), lambda p, i: (0, 0)),
            pl.BlockSpec((1, H), lambda p, i: (0, 0)),
        ],
        out_specs=[
            pl.BlockSpec((bm, H),
                         lambda p, i: (jnp.where(p < 2, 0,
                                                 jnp.where(p == 2, i,
                                                           ni - 1)), 0)),
            pl.BlockSpec((bm, H),
                         lambda p, i: (jnp.where(p < 3, 0, i), 0)),
        ],
        out_shape=[
            jax.ShapeDtypeStruct((N, H), jnp.float32),
            jax.ShapeDtypeStruct((N, H), jnp.float32),
        ],
        scratch_shapes=[
            pltpu.VMEM((N, N), jnp.bfloat16),
            pltpu.VMEM((N, F), jnp.bfloat16),
            pltpu.VMEM((N, F), jnp.bfloat16),
            pltpu.VMEM((N, H), jnp.bfloat16),
            pltpu.VMEM((N, H), jnp.bfloat16),
        ],
        compiler_params=pltpu.CompilerParams(
            vmem_limit_bytes=63 * 1024 * 1024,
        ),
    )(x1, x2, adj1, adj2,
      W11.astype(jnp.bfloat16), b11.reshape(1, F),
      W12.astype(jnp.bfloat16), b12.reshape(1, H),
      W21.astype(jnp.bfloat16), b21.reshape(1, F),
      W22.astype(jnp.bfloat16), b22.reshape(1, H))
    return z1, z2


def _sim_kernel(z1_ref, z2_ref, clm_ref, loss_ref, acc_ref):
    i = pl.program_id(0)
    z1 = z1_ref[...]
    z2 = z2_ref[...]
    r1 = jax.lax.rsqrt(jnp.sum(z1 * z1, axis=1, keepdims=True)) * (1.0 / TAU)
    r2 = jax.lax.rsqrt(jnp.sum(z2 * z2, axis=1, keepdims=True))
    s = jax.lax.dot_general(z1, z2, (((1,), (1,)), ((), ())),
                            preferred_element_type=jnp.float32)
    p = jnp.exp(s * r1 * r2.reshape(1, -1))
    rs = jnp.sum(p, axis=1, keepdims=True)
    ws = jnp.sum(p * clm_ref[...], axis=1, keepdims=True)
    part = jnp.sum(jnp.log(rs + 1e-8) - jnp.log(ws))

    @pl.when(i == 0)
    def _():
        acc_ref[0] = 0.0

    acc_ref[0] += part

    @pl.when(i == pl.num_programs(0) - 1)
    def _():
        loss_ref[...] = jnp.full((1, 1), acc_ref[0] * (1.0 / N),
                                 dtype=jnp.float32)


def _sim_loss(z1, z2, clm, bm=256):
    loss = pl.pallas_call(
        _sim_kernel,
        grid=(N // bm,),
        in_specs=[
            pl.BlockSpec((bm, H), lambda i: (i, 0)),
            pl.BlockSpec((N, H), lambda i: (0, 0)),
            pl.BlockSpec((bm, N), lambda i: (i, 0)),
        ],
        out_specs=pl.BlockSpec((1, 1), lambda i: (0, 0)),
        out_shape=jax.ShapeDtypeStruct((1, 1), jnp.float32),
        scratch_shapes=[pltpu.SMEM((1,), jnp.float32)],
    )(z1, z2, clm)
    return loss.reshape(())


def kernel(x1, adj1, x2, adj2, clm, W11, b11, W12, b12, W21, b21, W22, b22):
    z1, z2 = _towers(x1, x2, adj1, adj2,
                     W11, b11, W12, b12, W21, b21, W22, b22)
    loss = _sim_loss(z1, z2, clm)
    return (z1, z2, loss)


# speedup vs baseline: 1.5418x; 1.0493x over previous
import jax
import jax.numpy as jnp
from jax.experimental import pallas as pl
from jax.experimental.pallas import tpu as pltpu

N = 4096
F = 256
H = 128
TAU = 0.5


def _towers_kernel(x1_ref, x2_ref, adj1_ref, adj2_ref,
                   w11_ref, b11_ref, w12_ref, b12_ref,
                   w21_ref, b21_ref, w22_ref, b22_ref,
                   z1_ref, z2_ref,
                   adj_scr, sup1_scr, sup2_scr, s2a_scr, s2b_scr):
    p = pl.program_id(0)
    i = pl.program_id(1)
    bm = z1_ref.shape[0]

    @pl.when(p == 0)
    def _():
        xb1 = x1_ref[...].astype(jnp.bfloat16)
        sup = jnp.dot(xb1, w11_ref[...], preferred_element_type=jnp.float32)
        sup1_scr[pl.ds(i * bm, bm), :] = sup.astype(jnp.bfloat16)
        xb2 = x2_ref[...].astype(jnp.bfloat16)
        sup = jnp.dot(xb2, w21_ref[...], preferred_element_type=jnp.float32)
        sup2_scr[pl.ds(i * bm, bm), :] = sup.astype(jnp.bfloat16)

    @pl.when(p == 1)
    def _():
        ab = adj1_ref[...].astype(jnp.bfloat16)
        adj_scr[pl.ds(i * bm, bm), :] = ab
        acc = jnp.dot(ab, sup1_scr[...], preferred_element_type=jnp.float32)
        h = jnp.maximum(acc + b11_ref[...], 0.0)
        s2 = jnp.dot(h.astype(jnp.bfloat16), w12_ref[...],
                     preferred_element_type=jnp.float32)
        s2a_scr[pl.ds(i * bm, bm), :] = s2.astype(jnp.bfloat16)

    @pl.when(p == 2)
    def _():
        a1 = adj_scr[pl.ds(i * bm, bm), :]
        z1_ref[...] = jnp.dot(a1, s2a_scr[...],
                              preferred_element_type=jnp.float32) + b12_ref[...]
        ab = adj2_ref[...].astype(jnp.bfloat16)
        adj_scr[pl.ds(i * bm, bm), :] = ab
        acc = jnp.dot(ab, sup2_scr[...], preferred_element_type=jnp.float32)
        h = jnp.maximum(acc + b21_ref[...], 0.0)
        s2 = jnp.dot(h.astype(jnp.bfloat16), w22_ref[...],
                     preferred_element_type=jnp.float32)
        s2b_scr[pl.ds(i * bm, bm), :] = s2.astype(jnp.bfloat16)

    @pl.when(p == 3)
    def _():
        a2 = adj_scr[pl.ds(i * bm, bm), :]
        z2_ref[...] = jnp.dot(a2, s2b_scr[...],
                              preferred_element_type=jnp.float32) + b22_ref[...]


def _towers(x1, x2, adj1, adj2,
            W11, b11, W12, b12, W21, b21, W22, b22, bm=256):
    ni = N // bm
    z1, z2 = pl.pallas_call(
        _towers_kernel,
        grid=(4, ni),
        in_specs=[
            pl.BlockSpec((bm, F),
                         lambda p, i: (jnp.where(p == 0, i, ni - 1), 0)),
            pl.BlockSpec((bm, F),
                         lambda p, i: (jnp.where(p == 0, i, ni - 1), 0)),
            pl.BlockSpec((bm, N),
                         lambda p, i: (jnp.where(p < 1, 0,
                                                 jnp.where(p == 1, i,
                                                           ni - 1)), 0)),
            pl.BlockSpec((bm, N),
                         lambda p, i: (jnp.where(p < 2, 0,
                                                 jnp.where(p == 2, i,
                                                           ni - 1)), 0)),
            pl.BlockSpec((F, F), lambda p, i: (0, 0)),
            pl.BlockSpec((1, F), lambda p, i: (0, 0)),
            pl.BlockSpec((F, H), lambda p, i: (0, 0)),
            pl.BlockSpec((1, H), lambda p, i: (0, 0)),
            pl.BlockSpec((F, F), lambda p, i: (0, 0)),
            pl.BlockSpec((1, F), lambda p, i: (0, 0)),
            pl.BlockSpec((F, H), lambda p, i: (0, 0)),
            pl.BlockSpec((1, H), lambda p, i: (0, 0)),
        ],
        out_specs=[
            pl.BlockSpec((bm, H),
                         lambda p, i: (jnp.where(p < 2, 0,
                                                 jnp.where(p == 2, i,
                                                           ni - 1)), 0)),
            pl.BlockSpec((bm, H),
                         lambda p, i: (jnp.where(p < 3, 0, i), 0)),
        ],
        out_shape=[
            jax.ShapeDtypeStruct((N, H), jnp.float32),
            jax.ShapeDtypeStruct((N, H), jnp.float32),
        ],
        scratch_shapes=[
            pltpu.VMEM((N, N), jnp.bfloat16),
            pltpu.VMEM((N, F), jnp.bfloat16),
            pltpu.VMEM((N, F), jnp.bfloat16),
            pltpu.VMEM((N, H), jnp.bfloat16),
            pltpu.VMEM((N, H), jnp.bfloat16),
        ],
        compiler_params=pltpu.CompilerParams(
            vmem_limit_bytes=63 * 1024 * 1024,
        ),
    )(x1, x2, adj1, adj2,
      W11.astype(jnp.bfloat16), b11.reshape(1, F),
      W12.astype(jnp.bfloat16), b12.reshape(1, H),
      W21.astype(jnp.bfloat16), b21.reshape(1, F),
      W22.astype(jnp.bfloat16), b22.reshape(1, H))
    return z1, z2


def _sim_kernel(z1_ref, z2_ref, clm_ref, loss_ref, acc_ref):
    i = pl.program_id(0)
    z1 = z1_ref[...]
    z2 = z2_ref[...]
    r1 = jax.lax.rsqrt(jnp.sum(z1 * z1, axis=1, keepdims=True)) * (1.0 / TAU)
    r2 = jax.lax.rsqrt(jnp.sum(z2 * z2, axis=1, keepdims=True))
    s = jax.lax.dot_general(z1, z2, (((1,), (1,)), ((), ())),
                            preferred_element_type=jnp.float32)
    p = jnp.exp(s * r1 * r2.reshape(1, -1))
    rs = jnp.sum(p, axis=1, keepdims=True)
    ws = jnp.sum(p * clm_ref[...], axis=1, keepdims=True)
    part = jnp.sum(jnp.log(rs + 1e-8) - jnp.log(ws))

    @pl.when(i == 0)
    def _():
        acc_ref[0] = 0.0

    acc_ref[0] += part

    @pl.when(i == pl.num_programs(0) - 1)
    def _():
        loss_ref[...] = jnp.full((1, 1), acc_ref[0] * (1.0 / N),
                                 dtype=jnp.float32)


def _sim_loss(z1, z2, clm, bm=512):
    loss = pl.pallas_call(
        _sim_kernel,
        grid=(N // bm,),
        in_specs=[
            pl.BlockSpec((bm, H), lambda i: (i, 0)),
            pl.BlockSpec((N, H), lambda i: (0, 0)),
            pl.BlockSpec((bm, N), lambda i: (i, 0)),
        ],
        out_specs=pl.BlockSpec((1, 1), lambda i: (0, 0)),
        out_shape=jax.ShapeDtypeStruct((1, 1), jnp.float32),
        scratch_shapes=[pltpu.SMEM((1,), jnp.float32)],
    )(z1, z2, clm)
    return loss.reshape(())


def kernel(x1, adj1, x2, adj2, clm, W11, b11, W12, b12, W21, b21, W22, b22):
    z1, z2 = _towers(x1, x2, adj1, adj2,
                     W11, b11, W12, b12, W21, b21, W22, b22)
    loss = _sim_loss(z1, z2, clm)
    return (z1, z2, loss)
